# baseline (speedup 1.0000x reference)
import functools

import jax
import jax.numpy as jnp
from jax import lax
from jax.experimental import pallas as pl
from jax.experimental.pallas import tpu as pltpu
from jax.experimental.pallas import tpu_sc as plsc

N = 10000
N_PAD = 10240
E = 320000
D = 128
D_OUT = 64
NC = 2
NS = 16
NW = NC * NS
CHUNK = 128
GRP = 8
CHUNKS_W = 80
CHUNKS_T = 160
E_W = CHUNKS_W * CHUNK
E_PAD = NW * E_W
NROW = E_PAD // CHUNK
ROWS_T = N_PAD // NS
PAD_SRC = N
PAD_DST = N + 8

_mesh = plsc.VectorSubcoreMesh(core_axis_name="c", subcore_axis_name="s")
_mesh1 = plsc.VectorSubcoreMesh(core_axis_name="c", subcore_axis_name="s",
                                num_cores=1)


@functools.partial(
    pl.kernel,
    out_type=jax.ShapeDtypeStruct((NC, N_PAD, D), jnp.float32),
    mesh=_mesh,
    scratch_types=[
        pltpu.VMEM((GRP, CHUNK), jnp.int32),
        pltpu.VMEM((CHUNK, D), jnp.float32),
        pltpu.VMEM_SHARED((N_PAD, D), jnp.float32),
    ],
)
def _sc_deg(dst_hbm, ones_hbm, zeros_hbm, deg_out, idx_d, ones_v, dacc):
    c = lax.axis_index("c")
    s = lax.axis_index("s")
    wid = s * NC + c
    pltpu.sync_copy(ones_hbm, ones_v)
    r0 = s * ROWS_T
    pltpu.sync_copy(zeros_hbm, dacc.at[pl.ds(r0, ROWS_T)])
    plsc.subcore_barrier()
    row0 = wid * CHUNKS_W

    def grp_body(g, carry):
        pltpu.sync_copy(dst_hbm.at[pl.ds(row0 + g * GRP, GRP)], idx_d)

        def body(j, cr):
            pltpu.sync_copy(ones_v, dacc.at[idx_d.at[j]], add=True)
            return cr

        lax.fori_loop(0, GRP, body, carry)
        return carry

    lax.fori_loop(0, CHUNKS_W // GRP, grp_body, 0)
    plsc.subcore_barrier()
    pltpu.sync_copy(dacc.at[pl.ds(r0, ROWS_T)], deg_out.at[c, pl.ds(r0, ROWS_T)])


@functools.partial(
    pl.kernel,
    out_type=jax.ShapeDtypeStruct((N_PAD, D), jnp.float32),
    mesh=_mesh1,
    scratch_types=[
        pltpu.VMEM((2, GRP, CHUNK), jnp.int32),
        pltpu.VMEM((2, GRP, CHUNK), jnp.int32),
        pltpu.VMEM((2, CHUNK, D), jnp.float32),
        pltpu.VMEM_SHARED((N_PAD, D), jnp.float32),
        pltpu.SemaphoreType.DMA,
        pltpu.SemaphoreType.DMA,
    ],
)
def _sc_mp(g_hbm, src_hbm, dst_hbm, zeros_hbm, acc_out,
           ixs, ixd, rows2, acc, sem0, sem1):
    s = lax.axis_index("s")
    r0 = s * ROWS_T
    pltpu.sync_copy(zeros_hbm, acc.at[pl.ds(r0, ROWS_T)])
    plsc.subcore_barrier()
    nch = CHUNKS_T
    row0 = s * CHUNKS_T

    pltpu.sync_copy(src_hbm.at[pl.ds(row0, GRP)], ixs.at[0])
    pltpu.sync_copy(dst_hbm.at[pl.ds(row0, GRP)], ixd.at[0])
    pltpu.async_copy(g_hbm.at[ixs.at[0, 0]], rows2.at[0], sem0)
    pltpu.async_copy(g_hbm.at[ixs.at[0, 1]], rows2.at[1], sem1)

    def pair(i2, carry):
        for b in (0, 1):
            k = 2 * i2 + b
            sem = sem0 if b == 0 else sem1
            pltpu.make_async_copy(g_hbm.at[ixs.at[0, 0]], rows2.at[b], sem).wait()
            pltpu.sync_copy(rows2.at[b],
                            acc.at[ixd.at[(k // GRP) % 2, k % GRP]], add=True)
            nk = k + 2
            if b == 0:
                @pl.when(jnp.logical_and(nk % GRP == 0, nk < nch))
                def _():
                    ng = nk // GRP
                    pltpu.sync_copy(src_hbm.at[pl.ds(row0 + ng * GRP, GRP)],
                                    ixs.at[ng % 2])
                    pltpu.sync_copy(dst_hbm.at[pl.ds(row0 + ng * GRP, GRP)],
                                    ixd.at[ng % 2])

            @pl.when(nk < nch)
            def _():
                pltpu.async_copy(
                    g_hbm.at[ixs.at[(nk // GRP) % 2, nk % GRP]],
                    rows2.at[b], sem)

        return carry

    lax.fori_loop(0, nch // 2, pair, 0)
    plsc.subcore_barrier()
    pltpu.sync_copy(acc.at[pl.ds(r0, ROWS_T)], acc_out.at[pl.ds(r0, ROWS_T)])


def _mm(a, b):
    return lax.dot_general(a, b, (((1,), (0,)), ((), ())),
                           precision=lax.Precision.DEFAULT,
                           preferred_element_type=jnp.float32)


def _dinv(deg_ref):
    deg = deg_ref[0] + deg_ref[1]
    tot = deg[:, :1] + 1.0
    return lax.rsqrt(tot)


def _tc0_body(x_ref, w_ref, deg_ref, g_ref):
    g_ref[...] = _mm(x_ref[...], w_ref[...]) * _dinv(deg_ref)


def _tc0(x, w, deg2):
    return pl.pallas_call(
        _tc0_body,
        out_shape=jax.ShapeDtypeStruct((N_PAD, D), jnp.float32),
    )(x, w, deg2)


def _tcmid_body(acc_ref, g_ref, deg_ref, w_ref, b_ref, o_ref):
    dinv = _dinv(deg_ref)
    a = jnp.maximum(dinv * (acc_ref[...] + g_ref[...]) + b_ref[...], 0.0)
    o_ref[...] = _mm(a, w_ref[...]) * dinv


def _tcmid(acc2, g, deg2, w, b):
    return pl.pallas_call(
        _tcmid_body,
        out_shape=jax.ShapeDtypeStruct((N_PAD, D), jnp.float32),
    )(acc2, g, deg2, w, b)


def _tcfin_body(acc_ref, g_ref, deg_ref, b2_ref, qe_ref, fc0w_ref, fc0b_ref,
                batch_ref, w1x_ref, w1q_ref, fc1b_ref, fc2w_ref, fc2b_ref,
                o_ref):
    dinv = _dinv(deg_ref)
    h = jnp.maximum(dinv * (acc_ref[...] + g_ref[...]) + b2_ref[...], 0.0)
    q = jnp.maximum(_mm(qe_ref[...], fc0w_ref[...]) + fc0b_ref[...], 0.0)
    q2 = _mm(q, w1q_ref[...])
    oh = (batch_ref[...] ==
          lax.broadcasted_iota(jnp.int32, (1, 64), 1)).astype(jnp.float32)
    z = jnp.maximum(_mm(h, w1x_ref[...]) + _mm(oh, q2) + fc1b_ref[...], 0.0)
    o_ref[...] = _mm(z, fc2w_ref[...]) + fc2b_ref[...]


def _tcfin(acc2, g, deg2, b2, qe, fc0w, fc0b, batchp, w1x, w1q, fc1b, fc2w, fc2b):
    return pl.pallas_call(
        _tcfin_body,
        out_shape=jax.ShapeDtypeStruct((N_PAD, D_OUT), jnp.float32),
    )(acc2, g, deg2, b2, qe, fc0w, fc0b, batchp, w1x, w1q, fc1b, fc2w, fc2b)


def kernel(x, edge_index, batch, question_embedding, W0, b0, W1, b1, W2, b2,
           fc0_W, fc0_b, fc1_W, fc1_b, fc2_W, fc2_b):
    f32 = jnp.float32
    src = edge_index[0]
    dst = edge_index[1]
    srcp = jnp.concatenate(
        [src, jnp.full((E_PAD - E,), PAD_SRC, jnp.int32)]).reshape(NROW, CHUNK)
    dstp = jnp.concatenate(
        [dst, jnp.full((E_PAD - E,), PAD_DST, jnp.int32)]).reshape(NROW, CHUNK)
    xp = jnp.pad(x, ((0, N_PAD - N), (0, 0)))
    batchp = jnp.pad(batch, (0, N_PAD - N)).reshape(N_PAD, 1)
    onesD = jnp.ones((CHUNK, D), f32)
    zrows = jnp.zeros((ROWS_T, D), f32)

    deg2 = _sc_deg(dstp, onesD, zrows)
    g0 = _tc0(xp, W0, deg2)
    acc0 = _sc_mp(g0, srcp, dstp, zrows)
    g1 = _tcmid(acc0, g0, deg2, W1, b0.reshape(1, D))
    acc1 = _sc_mp(g1, srcp, dstp, zrows)
    g2 = _tcmid(acc1, g1, deg2, W2, b1.reshape(1, D))
    acc2 = _sc_mp(g2, srcp, dstp, zrows)
    out = _tcfin(acc2, g2, deg2, b2.reshape(1, D), question_embedding,
                 fc0_W, fc0_b.reshape(1, D), batchp,
                 fc1_W[:D], fc1_W[D:], fc1_b.reshape(1, D),
                 fc2_W, fc2_b.reshape(1, D_OUT))
    return out[:N]


# speedup vs baseline: 1.2283x; 1.2283x over previous
import functools

import jax
import jax.numpy as jnp
from jax import lax
from jax.experimental import pallas as pl
from jax.experimental.pallas import tpu as pltpu
from jax.experimental.pallas import tpu_sc as plsc

N = 10000
N_PAD = 10240
E = 320000
D = 128
D_OUT = 64
NC = 2
NS = 16
NW = NC * NS
CHUNK = 128
GRP = 8
CHUNKS_W = 80
CH_C0 = 120
CH_C1 = 40
E_W = CHUNKS_W * CHUNK
E_PAD = NW * E_W
NROW = E_PAD // CHUNK
ROWS_T = N_PAD // NS
PAD_SRC = N
PAD_DST = N + 8

_mesh = plsc.VectorSubcoreMesh(core_axis_name="c", subcore_axis_name="s")


@functools.partial(
    pl.kernel,
    out_type=jax.ShapeDtypeStruct((NC, N_PAD, D), jnp.float32),
    mesh=_mesh,
    scratch_types=[
        pltpu.VMEM((GRP, CHUNK), jnp.int32),
        pltpu.VMEM((CHUNK, D), jnp.float32),
        pltpu.VMEM_SHARED((N_PAD, D), jnp.float32),
    ],
)
def _sc_deg(dst_hbm, ones_hbm, zeros_hbm, deg_out, idx_d, ones_v, dacc):
    c = lax.axis_index("c")
    s = lax.axis_index("s")
    wid = s * NC + c
    pltpu.sync_copy(ones_hbm, ones_v)
    r0 = s * ROWS_T
    pltpu.sync_copy(zeros_hbm, dacc.at[pl.ds(r0, ROWS_T)])
    plsc.subcore_barrier()
    row0 = wid * CHUNKS_W

    def grp_body(g, carry):
        pltpu.sync_copy(dst_hbm.at[pl.ds(row0 + g * GRP, GRP)], idx_d)

        def body(j, cr):
            pltpu.sync_copy(ones_v, dacc.at[idx_d.at[j]], add=True)
            return cr

        lax.fori_loop(0, GRP, body, carry)
        return carry

    lax.fori_loop(0, CHUNKS_W // GRP, grp_body, 0)
    plsc.subcore_barrier()
    pltpu.sync_copy(dacc.at[pl.ds(r0, ROWS_T)], deg_out.at[c, pl.ds(r0, ROWS_T)])


@functools.partial(
    pl.kernel,
    out_type=jax.ShapeDtypeStruct((NC, N_PAD, D), jnp.float32),
    mesh=_mesh,
    scratch_types=[
        pltpu.VMEM((2, GRP, CHUNK), jnp.int32),
        pltpu.VMEM((2, GRP, CHUNK), jnp.int32),
        pltpu.VMEM((2, CHUNK, D), jnp.float32),
        pltpu.VMEM_SHARED((N_PAD, D), jnp.float32),
        pltpu.SemaphoreType.DMA,
        pltpu.SemaphoreType.DMA,
    ],
)
def _sc_mp(g_hbm, src_hbm, dst_hbm, zeros_hbm, acc_out,
           ixs, ixd, rows2, acc, sem0, sem1):
    c = lax.axis_index("c")
    s = lax.axis_index("s")
    r0 = s * ROWS_T
    pltpu.sync_copy(zeros_hbm, acc.at[pl.ds(r0, ROWS_T)])
    plsc.subcore_barrier()
    nch = jnp.where(c == 0, CH_C0, CH_C1)
    row0 = jnp.where(c == 0, s * CH_C0, NS * CH_C0 + s * CH_C1)

    pltpu.sync_copy(src_hbm.at[pl.ds(row0, GRP)], ixs.at[0])
    pltpu.sync_copy(dst_hbm.at[pl.ds(row0, GRP)], ixd.at[0])
    pltpu.async_copy(g_hbm.at[ixs.at[0, 0]], rows2.at[0], sem0)
    pltpu.async_copy(g_hbm.at[ixs.at[0, 1]], rows2.at[1], sem1)

    def pair(i2, carry):
        for b in (0, 1):
            k = 2 * i2 + b
            sem = sem0 if b == 0 else sem1
            pltpu.make_async_copy(g_hbm.at[ixs.at[0, 0]], rows2.at[b], sem).wait()
            pltpu.sync_copy(rows2.at[b],
                            acc.at[ixd.at[(k // GRP) % 2, k % GRP]], add=True)
            nk = k + 2
            if b == 0:
                @pl.when(jnp.logical_and(nk % GRP == 0, nk < nch))
                def _():
                    ng = nk // GRP
                    pltpu.sync_copy(src_hbm.at[pl.ds(row0 + ng * GRP, GRP)],
                                    ixs.at[ng % 2])
                    pltpu.sync_copy(dst_hbm.at[pl.ds(row0 + ng * GRP, GRP)],
                                    ixd.at[ng % 2])

            @pl.when(nk < nch)
            def _():
                pltpu.async_copy(
                    g_hbm.at[ixs.at[(nk // GRP) % 2, nk % GRP]],
                    rows2.at[b], sem)

        return carry

    lax.fori_loop(0, nch // 2, pair, 0)
    plsc.subcore_barrier()
    pltpu.sync_copy(acc.at[pl.ds(r0, ROWS_T)], acc_out.at[c, pl.ds(r0, ROWS_T)])


def _mm(a, b):
    return lax.dot_general(a, b, (((1,), (0,)), ((), ())),
                           precision=lax.Precision.DEFAULT,
                           preferred_element_type=jnp.float32)


def _dinv(deg_ref):
    deg = deg_ref[0] + deg_ref[1]
    tot = deg[:, :1] + 1.0
    return lax.rsqrt(tot)


def _tc0_body(x_ref, w_ref, deg_ref, g_ref):
    g_ref[...] = _mm(x_ref[...], w_ref[...]) * _dinv(deg_ref)


def _tc0(x, w, deg2):
    return pl.pallas_call(
        _tc0_body,
        out_shape=jax.ShapeDtypeStruct((N_PAD, D), jnp.float32),
    )(x, w, deg2)


def _tcmid_body(acc_ref, g_ref, deg_ref, w_ref, b_ref, o_ref):
    dinv = _dinv(deg_ref)
    a = jnp.maximum(dinv * (acc_ref[0] + acc_ref[1] + g_ref[...]) + b_ref[...], 0.0)
    o_ref[...] = _mm(a, w_ref[...]) * dinv


def _tcmid(acc2, g, deg2, w, b):
    return pl.pallas_call(
        _tcmid_body,
        out_shape=jax.ShapeDtypeStruct((N_PAD, D), jnp.float32),
    )(acc2, g, deg2, w, b)


def _tcfin_body(acc_ref, g_ref, deg_ref, b2_ref, qe_ref, fc0w_ref, fc0b_ref,
                batch_ref, w1x_ref, w1q_ref, fc1b_ref, fc2w_ref, fc2b_ref,
                o_ref):
    dinv = _dinv(deg_ref)
    h = jnp.maximum(dinv * (acc_ref[0] + acc_ref[1] + g_ref[...]) + b2_ref[...], 0.0)
    q = jnp.maximum(_mm(qe_ref[...], fc0w_ref[...]) + fc0b_ref[...], 0.0)
    q2 = _mm(q, w1q_ref[...])
    oh = (batch_ref[...] ==
          lax.broadcasted_iota(jnp.int32, (1, 64), 1)).astype(jnp.float32)
    z = jnp.maximum(_mm(h, w1x_ref[...]) + _mm(oh, q2) + fc1b_ref[...], 0.0)
    o_ref[...] = _mm(z, fc2w_ref[...]) + fc2b_ref[...]


def _tcfin(acc2, g, deg2, b2, qe, fc0w, fc0b, batchp, w1x, w1q, fc1b, fc2w, fc2b):
    return pl.pallas_call(
        _tcfin_body,
        out_shape=jax.ShapeDtypeStruct((N_PAD, D_OUT), jnp.float32),
    )(acc2, g, deg2, b2, qe, fc0w, fc0b, batchp, w1x, w1q, fc1b, fc2w, fc2b)


def kernel(x, edge_index, batch, question_embedding, W0, b0, W1, b1, W2, b2,
           fc0_W, fc0_b, fc1_W, fc1_b, fc2_W, fc2_b):
    f32 = jnp.float32
    src = edge_index[0]
    dst = edge_index[1]
    srcp = jnp.concatenate(
        [src, jnp.full((E_PAD - E,), PAD_SRC, jnp.int32)]).reshape(NROW, CHUNK)
    dstp = jnp.concatenate(
        [dst, jnp.full((E_PAD - E,), PAD_DST, jnp.int32)]).reshape(NROW, CHUNK)
    xp = jnp.pad(x, ((0, N_PAD - N), (0, 0)))
    batchp = jnp.pad(batch, (0, N_PAD - N)).reshape(N_PAD, 1)
    onesD = jnp.ones((CHUNK, D), f32)
    zrows = jnp.zeros((ROWS_T, D), f32)

    deg2 = _sc_deg(dstp, onesD, zrows)
    g0 = _tc0(xp, W0, deg2)
    acc0 = _sc_mp(g0, srcp, dstp, zrows)
    g1 = _tcmid(acc0, g0, deg2, W1, b0.reshape(1, D))
    acc1 = _sc_mp(g1, srcp, dstp, zrows)
    g2 = _tcmid(acc1, g1, deg2, W2, b1.reshape(1, D))
    acc2 = _sc_mp(g2, srcp, dstp, zrows)
    out = _tcfin(acc2, g2, deg2, b2.reshape(1, D), question_embedding,
                 fc0_W, fc0_b.reshape(1, D), batchp,
                 fc1_W[:D], fc1_W[D:], fc1_b.reshape(1, D),
                 fc2_W, fc2_b.reshape(1, D_OUT))
    return out[:N]
